# baseline (speedup 1.0000x reference)
import functools

import jax
import jax.numpy as jnp
from jax.experimental import pallas as pl
from jax.experimental.pallas import tpu as pltpu

_BN = 1024
_BJ = 512
_NJF = 8


def _lstm_kernel(hb_ref, uf_ref, ui_ref, x_ref, wf_ref, wi_ref, nc_ref,
                 mask_ref, bf_ref, bi_ref, buf_ref, bui_ref,
                 h_out, c_out,
                 fin_ref, xiou_ref, cagg_ref, ig_ref, og_ref, bsl_ref,
                 *, h_size):
    j = pl.program_id(1)

    @pl.when(j == 0)
    def _():
        mask = mask_ref[...]
        xb = x_ref[...].astype(jnp.bfloat16)
        fin_ref[...] = (jnp.dot(xb, wf_ref[...],
                                preferred_element_type=jnp.float32)
                        + bf_ref[...]) * mask
        xiou_ref[...] = (jnp.dot(xb, wi_ref[...],
                                 preferred_element_type=jnp.float32)
                         + bi_ref[...]) * mask
        cagg_ref[...] = jnp.zeros_like(cagg_ref)

    for jj in range(_NJF):
        @pl.when(j == jj)
        def _(jj=jj):
            bsl_ref[...] = buf_ref[:, jj * _BJ:(jj + 1) * _BJ]

    @pl.when(j < _NJF)
    def _():
        res = jnp.dot(hb_ref[...], uf_ref[...],
                      preferred_element_type=jnp.float32)
        fin = fin_ref[...]
        f0 = jax.nn.sigmoid(res[:, :h_size] + bsl_ref[:, :h_size] + fin)
        f1 = jax.nn.sigmoid(res[:, h_size:] + bsl_ref[:, h_size:] + fin)
        cagg_ref[...] += (f0 * nc_ref[:, :h_size]
                          + f1 * nc_ref[:, h_size:])

    @pl.when(j == _NJF)
    def _():
        res = jnp.dot(hb_ref[...], ui_ref[...],
                      preferred_element_type=jnp.float32)
        ig_ref[...] = jax.nn.sigmoid(res[:, :h_size] + bui_ref[:, :h_size]
                                     + xiou_ref[:, :h_size])
        og_ref[...] = jax.nn.sigmoid(
            res[:, h_size:] + bui_ref[:, h_size:2 * h_size]
            + xiou_ref[:, h_size:2 * h_size])

    @pl.when(j == _NJF + 1)
    def _():
        res = jnp.dot(hb_ref[...], ui_ref[...],
                      preferred_element_type=jnp.float32)
        u = jnp.tanh(res[:, :h_size] + bui_ref[:, 2 * h_size:3 * h_size]
                     + xiou_ref[:, 2 * h_size:])
        c = ig_ref[...] * u + cagg_ref[...]
        h_out[...] = og_ref[...] * jnp.tanh(c)
        c_out[...] = c


def kernel(x, x_mask, neighbour_h, neighbour_c, W_iou, b_iou, W_f, b_f,
           U_iou, bU_iou, U_f, bU_f):
    n, n_ch, h_size = neighbour_h.shape
    x_size = x.shape[1]
    kdim = n_ch * h_size
    nj = _NJF + 2

    hb = neighbour_h.reshape(n, kdim).astype(jnp.bfloat16)
    ncb = neighbour_c.reshape(n, kdim).astype(jnp.bfloat16)
    uf16 = U_f[:, :kdim].astype(jnp.bfloat16)
    ui16 = jnp.pad(U_iou, ((0, 0), (0, 2 * _BJ - 3 * h_size))
                   ).astype(jnp.bfloat16)
    wf16 = W_f.astype(jnp.bfloat16)
    wi16 = W_iou.astype(jnp.bfloat16)
    mask = x_mask[:, None]
    bf2 = b_f[None, :]
    bi2 = b_iou[None, :]
    buf2 = bU_f[None, :kdim]
    bui2 = bU_iou[None, :]

    ni = pl.cdiv(n, _BN)
    out_shape = (
        jax.ShapeDtypeStruct((n, h_size), jnp.float32),
        jax.ShapeDtypeStruct((n, h_size), jnp.float32),
    )
    h_out, c_out = pl.pallas_call(
        functools.partial(_lstm_kernel, h_size=h_size),
        grid=(ni, nj),
        in_specs=[
            pl.BlockSpec((_BN, kdim), lambda i, j: (i, 0)),
            pl.BlockSpec((kdim, _BJ),
                         lambda i, j: (0, jnp.minimum(j, _NJF - 1))),
            pl.BlockSpec((kdim, _BJ),
                         lambda i, j: (0, jnp.maximum(j - _NJF, 0))),
            pl.BlockSpec((_BN, x_size), lambda i, j: (i, 0)),
            pl.BlockSpec((x_size, h_size), lambda i, j: (0, 0)),
            pl.BlockSpec((x_size, 3 * h_size), lambda i, j: (0, 0)),
            pl.BlockSpec((_BN, _BJ),
                         lambda i, j: (i, jnp.minimum(j, _NJF - 1))),
            pl.BlockSpec((_BN, 1), lambda i, j: (i, 0)),
            pl.BlockSpec((1, h_size), lambda i, j: (0, 0)),
            pl.BlockSpec((1, 3 * h_size), lambda i, j: (0, 0)),
            pl.BlockSpec((1, kdim), lambda i, j: (0, 0)),
            pl.BlockSpec((1, 3 * h_size), lambda i, j: (0, 0)),
        ],
        out_specs=(
            pl.BlockSpec((_BN, h_size), lambda i, j: (i, 0)),
            pl.BlockSpec((_BN, h_size), lambda i, j: (i, 0)),
        ),
        out_shape=out_shape,
        scratch_shapes=[
            pltpu.VMEM((_BN, h_size), jnp.float32),
            pltpu.VMEM((_BN, 3 * h_size), jnp.float32),
            pltpu.VMEM((_BN, h_size), jnp.float32),
            pltpu.VMEM((_BN, h_size), jnp.float32),
            pltpu.VMEM((_BN, h_size), jnp.float32),
            pltpu.VMEM((1, _BJ), jnp.float32),
        ],
        compiler_params=pltpu.CompilerParams(
            dimension_semantics=("arbitrary", "arbitrary"),
        ),
    )(hb, uf16, ui16, x, wf16, wi16, ncb, mask, bf2, bi2, buf2, bui2)
    return h_out, c_out


# speedup vs baseline: 1.2920x; 1.2920x over previous
import functools

import jax
import jax.numpy as jnp
from jax.experimental import pallas as pl
from jax.experimental.pallas import tpu as pltpu

_BN = 512
_BJ = 512
_NJF = 8


def _mailbox_copies(src_ref, dst_ref, sem, row0, nrows, n_ch, h_size):
    return [
        pltpu.make_async_copy(
            src_ref.at[pl.ds(row0, nrows), ch, :],
            dst_ref.at[pl.ds(0, nrows), pl.ds(ch * h_size, h_size)],
            sem)
        for ch in range(n_ch)
    ]


def _lstm_kernel(h3_ref, nc3_ref, uf_ref, ui_ref, x_ref, wf_ref, wi_ref,
                 mask_ref, bf_ref, bi_ref, buf_ref, bui_ref,
                 h_out, c_out,
                 hbf_ref, hb_ref, ncf_ref, fin_ref, cagg_ref, ig_ref,
                 og_ref, h_sem, nc_sem,
                 *, ni, n_ch, h_size, n_rows, last_rows):
    i = pl.program_id(0)
    j = pl.program_id(1)
    full = _BN

    def h_copies(blk, nrows):
        return _mailbox_copies(h3_ref, hbf_ref, h_sem, blk * _BN, nrows,
                               n_ch, h_size)

    def nc_copies(blk, nrows):
        return _mailbox_copies(nc3_ref, ncf_ref, nc_sem, blk * _BN, nrows,
                               n_ch, h_size)

    @pl.when(jnp.logical_and(i == 0, j == 0))
    def _():
        for c in h_copies(0, full):
            c.start()
        for c in nc_copies(0, full):
            c.start()

    @pl.when(j == 0)
    def _():
        @pl.when(i == ni - 1)
        def _():
            for c in h_copies(i, last_rows):
                c.wait()
            for c in nc_copies(i, last_rows):
                c.wait()

        @pl.when(i < ni - 1)
        def _():
            for c in h_copies(i, full):
                c.wait()
            for c in nc_copies(i, full):
                c.wait()

        hb_ref[...] = hbf_ref[...].astype(jnp.bfloat16)
        fin_ref[...] = (
            jnp.dot(x_ref[...].astype(jnp.bfloat16), wf_ref[...],
                    preferred_element_type=jnp.float32)
            + bf_ref[...]) * mask_ref[...]
        cagg_ref[...] = jnp.zeros_like(cagg_ref)

    @pl.when(jnp.logical_and(j == 1, i + 1 < ni))
    def _():
        @pl.when(i + 1 == ni - 1)
        def _():
            for c in h_copies(i + 1, last_rows):
                c.start()

        @pl.when(i + 1 < ni - 1)
        def _():
            for c in h_copies(i + 1, full):
                c.start()

    for jj in range(_NJF):
        @pl.when(j == jj)
        def _(jj=jj):
            res = jnp.dot(hb_ref[...], uf_ref[...],
                          preferred_element_type=jnp.float32)
            fin = fin_ref[...]
            lo = jj * _BJ
            f0 = jax.nn.sigmoid(res[:, :h_size]
                                + buf_ref[:, lo:lo + h_size] + fin)
            f1 = jax.nn.sigmoid(res[:, h_size:]
                                + buf_ref[:, lo + h_size:lo + 2 * h_size]
                                + fin)
            cagg_ref[...] += (f0 * ncf_ref[:, lo:lo + h_size]
                              + f1 * ncf_ref[:, lo + h_size:lo + 2 * h_size])

    @pl.when(jnp.logical_and(j == _NJF, i + 1 < ni))
    def _():
        @pl.when(i + 1 == ni - 1)
        def _():
            for c in nc_copies(i + 1, last_rows):
                c.start()

        @pl.when(i + 1 < ni - 1)
        def _():
            for c in nc_copies(i + 1, full):
                c.start()

    @pl.when(j == _NJF)
    def _():
        res = jnp.dot(hb_ref[...], ui_ref[...],
                      preferred_element_type=jnp.float32)
        xio = (jnp.dot(x_ref[...].astype(jnp.bfloat16), wi_ref[:, :2 * h_size],
                       preferred_element_type=jnp.float32)
               + bi_ref[:, :2 * h_size]) * mask_ref[...]
        ig_ref[...] = jax.nn.sigmoid(res[:, :h_size] + bui_ref[:, :h_size]
                                     + xio[:, :h_size])
        og_ref[...] = jax.nn.sigmoid(
            res[:, h_size:] + bui_ref[:, h_size:2 * h_size]
            + xio[:, h_size:])

    @pl.when(j == _NJF + 1)
    def _():
        res = jnp.dot(hb_ref[...], ui_ref[...],
                      preferred_element_type=jnp.float32)
        xu = (jnp.dot(x_ref[...].astype(jnp.bfloat16),
                      wi_ref[:, 2 * h_size:],
                      preferred_element_type=jnp.float32)
              + bi_ref[:, 2 * h_size:]) * mask_ref[...]
        u = jnp.tanh(res[:, :h_size] + bui_ref[:, 2 * h_size:3 * h_size]
                     + xu)
        c = ig_ref[...] * u + cagg_ref[...]
        h_out[...] = og_ref[...] * jnp.tanh(c)
        c_out[...] = c


def kernel(x, x_mask, neighbour_h, neighbour_c, W_iou, b_iou, W_f, b_f,
           U_iou, bU_iou, U_f, bU_f):
    n, n_ch, h_size = neighbour_h.shape
    x_size = x.shape[1]
    kdim = n_ch * h_size
    nj = _NJF + 2
    ni = pl.cdiv(n, _BN)
    last_rows = n - (ni - 1) * _BN

    uf16 = U_f[:, :kdim].astype(jnp.bfloat16)
    ui16 = jnp.pad(U_iou, ((0, 0), (0, 2 * _BJ - 3 * h_size))
                   ).astype(jnp.bfloat16)
    wf16 = W_f.astype(jnp.bfloat16)
    wi16 = W_iou.astype(jnp.bfloat16)
    mask = x_mask[:, None]
    bf2 = b_f[None, :]
    bi2 = b_iou[None, :]
    buf2 = bU_f[None, :kdim]
    bui2 = bU_iou[None, :]

    out_shape = (
        jax.ShapeDtypeStruct((n, h_size), jnp.float32),
        jax.ShapeDtypeStruct((n, h_size), jnp.float32),
    )
    h_out, c_out = pl.pallas_call(
        functools.partial(_lstm_kernel, ni=ni, n_ch=n_ch, h_size=h_size,
                          n_rows=n, last_rows=last_rows),
        grid=(ni, nj),
        in_specs=[
            pl.BlockSpec(memory_space=pltpu.MemorySpace.HBM),
            pl.BlockSpec(memory_space=pltpu.MemorySpace.HBM),
            pl.BlockSpec((kdim, _BJ),
                         lambda i, j: (0, jnp.minimum(j, _NJF - 1))),
            pl.BlockSpec((kdim, _BJ),
                         lambda i, j: (0, jnp.maximum(j - _NJF, 0))),
            pl.BlockSpec((_BN, x_size), lambda i, j: (i, 0)),
            pl.BlockSpec((x_size, h_size), lambda i, j: (0, 0)),
            pl.BlockSpec((x_size, 3 * h_size), lambda i, j: (0, 0)),
            pl.BlockSpec((_BN, 1), lambda i, j: (i, 0)),
            pl.BlockSpec((1, h_size), lambda i, j: (0, 0)),
            pl.BlockSpec((1, 3 * h_size), lambda i, j: (0, 0)),
            pl.BlockSpec((1, kdim), lambda i, j: (0, 0)),
            pl.BlockSpec((1, 3 * h_size), lambda i, j: (0, 0)),
        ],
        out_specs=(
            pl.BlockSpec((_BN, h_size), lambda i, j: (i, 0)),
            pl.BlockSpec((_BN, h_size), lambda i, j: (i, 0)),
        ),
        out_shape=out_shape,
        scratch_shapes=[
            pltpu.VMEM((_BN, kdim), jnp.float32),
            pltpu.VMEM((_BN, kdim), jnp.bfloat16),
            pltpu.VMEM((_BN, kdim), jnp.float32),
            pltpu.VMEM((_BN, h_size), jnp.float32),
            pltpu.VMEM((_BN, h_size), jnp.float32),
            pltpu.VMEM((_BN, h_size), jnp.float32),
            pltpu.VMEM((_BN, h_size), jnp.float32),
            pltpu.SemaphoreType.DMA,
            pltpu.SemaphoreType.DMA,
        ],
        compiler_params=pltpu.CompilerParams(
            dimension_semantics=("arbitrary", "arbitrary"),
        ),
    )(neighbour_h, neighbour_c, uf16, ui16, x, wf16, wi16, mask,
      bf2, bi2, buf2, bui2)
    return h_out, c_out


# speedup vs baseline: 1.4392x; 1.1140x over previous
import functools

import jax
import jax.numpy as jnp
from jax.experimental import pallas as pl
from jax.experimental.pallas import tpu as pltpu

_BN = 256
_BJ = 512
_NJF = 8


def _mailbox_copies(src_ref, dst_ref, sem, row0, nrows, n_ch, h_size):
    return [
        pltpu.make_async_copy(
            src_ref.at[pl.ds(row0, nrows), ch, :],
            dst_ref.at[pl.ds(0, nrows), pl.ds(ch * h_size, h_size)],
            sem)
        for ch in range(n_ch)
    ]


def _lstm_kernel(h3_ref, nc3_ref, ufh_ref, uih_ref, x_ref, wf_ref, wi_ref,
                 mask_ref, bf_ref, bi_ref, buf_ref, bui_ref,
                 h_out, c_out,
                 hbf_ref, hb_ref, ncf_ref, uf_ref, ui_ref,
                 h_sem, nc_sem, w_sem,
                 *, ni, n_ch, h_size, last_rows):
    i = pl.program_id(0)

    def h_copies(blk, nrows):
        return _mailbox_copies(h3_ref, hbf_ref, h_sem, blk * _BN, nrows,
                               n_ch, h_size)

    def nc_copies(blk, nrows):
        return _mailbox_copies(nc3_ref, ncf_ref, nc_sem, blk * _BN, nrows,
                               n_ch, h_size)

    def w_copies():
        return [pltpu.make_async_copy(ufh_ref, uf_ref, w_sem),
                pltpu.make_async_copy(uih_ref, ui_ref, w_sem)]

    @pl.when(i == 0)
    def _():
        for c in w_copies():
            c.start()
        for c in h_copies(0, _BN):
            c.start()
        for c in nc_copies(0, _BN):
            c.start()
        for c in w_copies():
            c.wait()

    @pl.when(i == ni - 1)
    def _():
        for c in h_copies(i, last_rows) + nc_copies(i, last_rows):
            c.wait()

    @pl.when(i < ni - 1)
    def _():
        for c in h_copies(i, _BN) + nc_copies(i, _BN):
            c.wait()

    hb_ref[...] = hbf_ref[...].astype(jnp.bfloat16)
    mask = mask_ref[...]
    xb = x_ref[...]
    fin = (jnp.dot(xb, wf_ref[...], preferred_element_type=jnp.float32)
           + bf_ref[...]) * mask

    @pl.when(i + 1 == ni - 1)
    def _():
        for c in h_copies(i + 1, last_rows):
            c.start()

    @pl.when(i + 1 < ni - 1)
    def _():
        for c in h_copies(i + 1, _BN):
            c.start()

    hb = hb_ref[...]

    def f_epilogue(t, res, cagg):
        lo = t * _BJ
        f0 = jax.nn.sigmoid(res[:, :h_size]
                            + buf_ref[:, lo:lo + h_size] + fin)
        f1 = jax.nn.sigmoid(res[:, h_size:]
                            + buf_ref[:, lo + h_size:lo + 2 * h_size] + fin)
        return (cagg + f0 * ncf_ref[:, lo:lo + h_size]
                + f1 * ncf_ref[:, lo + h_size:lo + 2 * h_size])

    cagg = jnp.zeros((_BN, h_size), dtype=jnp.float32)
    res_prev = jnp.dot(hb, uf_ref[:, :_BJ],
                       preferred_element_type=jnp.float32)
    for t in range(1, _NJF):
        res_t = jnp.dot(hb, uf_ref[:, t * _BJ:(t + 1) * _BJ],
                        preferred_element_type=jnp.float32)
        cagg = f_epilogue(t - 1, res_prev, cagg)
        res_prev = res_t
    riou = jnp.dot(hb, ui_ref[...], preferred_element_type=jnp.float32)
    cagg = f_epilogue(_NJF - 1, res_prev, cagg)

    @pl.when(i + 1 == ni - 1)
    def _():
        for c in nc_copies(i + 1, last_rows):
            c.start()

    @pl.when(i + 1 < ni - 1)
    def _():
        for c in nc_copies(i + 1, _BN):
            c.start()

    xio = (jnp.dot(xb, wi_ref[...], preferred_element_type=jnp.float32)
           + bi_ref[...]) * mask
    iou = riou + bui_ref[...] + xio
    i_g = jax.nn.sigmoid(iou[:, :h_size])
    o_g = jax.nn.sigmoid(iou[:, h_size:2 * h_size])
    u_g = jnp.tanh(iou[:, 2 * h_size:])
    c = i_g * u_g + cagg
    h_out[...] = o_g * jnp.tanh(c)
    c_out[...] = c


def kernel(x, x_mask, neighbour_h, neighbour_c, W_iou, b_iou, W_f, b_f,
           U_iou, bU_iou, U_f, bU_f):
    n, n_ch, h_size = neighbour_h.shape
    x_size = x.shape[1]
    kdim = n_ch * h_size
    ni = pl.cdiv(n, _BN)
    last_rows = n - (ni - 1) * _BN

    uf16 = U_f[:, :kdim].astype(jnp.bfloat16)
    ui16 = U_iou.astype(jnp.bfloat16)
    wf16 = W_f.astype(jnp.bfloat16)
    wi16 = W_iou.astype(jnp.bfloat16)
    x16 = x.astype(jnp.bfloat16)
    mask = x_mask[:, None]
    bf2 = b_f[None, :]
    bi2 = b_iou[None, :]
    buf2 = bU_f[None, :kdim]
    bui2 = bU_iou[None, :]

    out_shape = (
        jax.ShapeDtypeStruct((n, h_size), jnp.float32),
        jax.ShapeDtypeStruct((n, h_size), jnp.float32),
    )
    h_out, c_out = pl.pallas_call(
        functools.partial(_lstm_kernel, ni=ni, n_ch=n_ch, h_size=h_size,
                          last_rows=last_rows),
        grid=(ni,),
        in_specs=[
            pl.BlockSpec(memory_space=pltpu.MemorySpace.HBM),
            pl.BlockSpec(memory_space=pltpu.MemorySpace.HBM),
            pl.BlockSpec(memory_space=pltpu.MemorySpace.HBM),
            pl.BlockSpec(memory_space=pltpu.MemorySpace.HBM),
            pl.BlockSpec((_BN, x_size), lambda i: (i, 0)),
            pl.BlockSpec((x_size, h_size), lambda i: (0, 0)),
            pl.BlockSpec((x_size, 3 * h_size), lambda i: (0, 0)),
            pl.BlockSpec((_BN, 1), lambda i: (i, 0)),
            pl.BlockSpec((1, h_size), lambda i: (0, 0)),
            pl.BlockSpec((1, 3 * h_size), lambda i: (0, 0)),
            pl.BlockSpec((1, kdim), lambda i: (0, 0)),
            pl.BlockSpec((1, 3 * h_size), lambda i: (0, 0)),
        ],
        out_specs=(
            pl.BlockSpec((_BN, h_size), lambda i: (i, 0)),
            pl.BlockSpec((_BN, h_size), lambda i: (i, 0)),
        ),
        out_shape=out_shape,
        scratch_shapes=[
            pltpu.VMEM((_BN, kdim), jnp.float32),
            pltpu.VMEM((_BN, kdim), jnp.bfloat16),
            pltpu.VMEM((_BN, kdim), jnp.float32),
            pltpu.VMEM((kdim, kdim), jnp.bfloat16),
            pltpu.VMEM((kdim, 3 * h_size), jnp.bfloat16),
            pltpu.SemaphoreType.DMA,
            pltpu.SemaphoreType.DMA,
            pltpu.SemaphoreType.DMA,
        ],
        compiler_params=pltpu.CompilerParams(
            dimension_semantics=("arbitrary",),
        ),
    )(neighbour_h, neighbour_c, uf16, ui16, x16, wf16, wi16, mask,
      bf2, bi2, buf2, bui2)
    return h_out, c_out
